# baseline (speedup 1.0000x reference)
import functools

import jax
import jax.numpy as jnp
from jax import lax
from jax.experimental import pallas as pl
from jax.experimental.pallas import tpu as pltpu
from jax.experimental.pallas import tpu_sc as plsc

_NC = 2
_NS = 16
_NW = _NC * _NS
_LANES = 16


def _sc_body(nchunk, ch, wpe_hbm, wte_hbm, pos_hbm, idx_hbm, out_hbm,
             pos_v, idx_v, out_v, wpe_v, wte_v, tab_v):
    wid = lax.axis_index("s") * _NC + lax.axis_index("c")
    per_w = nchunk * ch

    pltpu.sync_copy(wpe_hbm, wpe_v)
    pltpu.sync_copy(wte_hbm, wte_v)
    lane = lax.iota(jnp.int32, _LANES)
    p_i = lane // 3
    i_i = lane - p_i * 3
    tab_v[...] = (plsc.load_gather(wpe_v, [p_i])
                  + plsc.load_gather(wte_v, [i_i]))

    base_w = wid * per_w
    nvec = ch // _LANES

    def chunk_body(ci, carry):
        base = base_w + ci * ch
        pltpu.sync_copy(pos_hbm.at[pl.ds(base, ch)], pos_v)
        pltpu.sync_copy(idx_hbm.at[pl.ds(base, ch)], idx_v)

        def vec_body(vi, c2):
            o = vi * _LANES
            fused = pos_v[pl.ds(o, _LANES)] * 3 + idx_v[pl.ds(o, _LANES)]
            out_v[pl.ds(o, _LANES)] = plsc.load_gather(tab_v, [fused])
            return c2

        lax.fori_loop(0, nvec, vec_body, 0)
        pltpu.sync_copy(out_v, out_hbm.at[pl.ds(base, ch)])
        return carry

    lax.fori_loop(0, nchunk, chunk_body, 0)


@functools.partial(jax.jit, static_argnums=(4, 5))
def _sc_lookup(wpe16, wte16, pos_f, idx_f, nchunk, ch):
    n = pos_f.shape[0]
    mesh = plsc.VectorSubcoreMesh(core_axis_name="c", subcore_axis_name="s")
    fn = pl.kernel(
        functools.partial(_sc_body, nchunk, ch),
        out_type=jax.ShapeDtypeStruct((n,), jnp.float32),
        mesh=mesh,
        scratch_types=[
            pltpu.VMEM((ch,), jnp.int32),
            pltpu.VMEM((ch,), jnp.int32),
            pltpu.VMEM((ch,), jnp.float32),
            pltpu.VMEM((_LANES,), jnp.float32),
            pltpu.VMEM((_LANES,), jnp.float32),
            pltpu.VMEM((_LANES,), jnp.float32),
        ],
        compiler_params=pltpu.CompilerParams(needs_layout_passes=False),
    )
    return fn(wpe16, wte16, pos_f, idx_f)


def kernel(pos, idx, wpe, wte):
    b, l = pos.shape
    n = b * l
    per_w = n // _NW
    assert per_w * _NW == n
    ch = per_w
    nchunk = 1
    while 3 * ch * 4 > 360 * 1024:
        for d in (2, 5, 3, 7):
            if ch % d == 0:
                ch //= d
                nchunk *= d
                break
        else:
            raise ValueError("no valid chunking")
    assert ch % _LANES == 0

    pos_f = pos.reshape(-1)
    idx_f = idx.reshape(-1)
    wpe16 = jnp.pad(wpe.reshape(-1), (0, _LANES - wpe.size))
    wte16 = jnp.pad(wte.reshape(-1), (0, _LANES - wte.size))
    out = _sc_lookup(wpe16, wte16, pos_f, idx_f, nchunk, ch)
    return out.reshape(b, l, 1)


# speedup vs baseline: 1.1251x; 1.1251x over previous
import functools

import jax
import jax.numpy as jnp
from jax import lax
from jax.experimental import pallas as pl
from jax.experimental.pallas import tpu as pltpu
from jax.experimental.pallas import tpu_sc as plsc

_NC = 2
_NS = 16
_NW = _NC * _NS
_LANES = 16
_UNROLL = 8


def _sc_body(nchunk, ch, wpe_hbm, wte_hbm, pos_hbm, idx_hbm, out_hbm,
             pos_v, idx_v, out_v, wpe_v, wte_v, tab_v,
             is0, is1, os0, os1):
    wid = lax.axis_index("s") * _NC + lax.axis_index("c")
    per_w = nchunk * ch
    base_w = wid * per_w
    isems = (is0, is1)
    osems = (os0, os1)

    pltpu.sync_copy(wpe_hbm, wpe_v)
    pltpu.sync_copy(wte_hbm, wte_v)
    lane = lax.iota(jnp.int32, _LANES)
    p_i = lane // 3
    i_i = lane - p_i * 3
    tab_v[...] = (plsc.load_gather(wpe_v, [p_i])
                  + plsc.load_gather(wte_v, [i_i]))

    def in_copies(ci, buf):
        base = base_w + ci * ch
        return (
            pltpu.make_async_copy(pos_hbm.at[pl.ds(base, ch)],
                                  pos_v.at[buf], isems[buf]),
            pltpu.make_async_copy(idx_hbm.at[pl.ds(base, ch)],
                                  idx_v.at[buf], isems[buf]),
        )

    def out_copy(ci, buf):
        base = base_w + ci * ch
        return pltpu.make_async_copy(out_v.at[buf],
                                     out_hbm.at[pl.ds(base, ch)], osems[buf])

    def compute(buf):
        @plsc.parallel_loop(0, ch, step=_LANES, unroll=_UNROLL)
        def _(o):
            fused = (pos_v[buf, pl.ds(o, _LANES)] * 3
                     + idx_v[buf, pl.ds(o, _LANES)])
            out_v[buf, pl.ds(o, _LANES)] = plsc.load_gather(tab_v, [fused])

    for c in in_copies(0, 0):
        c.start()
    for ci in range(nchunk):
        buf = ci % 2
        if ci + 1 < nchunk:
            for c in in_copies(ci + 1, 1 - buf):
                c.start()
        for c in in_copies(ci, buf):
            c.wait()
        if ci >= 2:
            out_copy(ci - 2, buf).wait()
        compute(buf)
        out_copy(ci, buf).start()
    for ci in range(max(nchunk - 2, 0), nchunk):
        out_copy(ci, ci % 2).wait()


@functools.partial(jax.jit, static_argnums=(4, 5))
def _sc_lookup(wpe16, wte16, pos_f, idx_f, nchunk, ch):
    n = pos_f.shape[0]
    mesh = plsc.VectorSubcoreMesh(core_axis_name="c", subcore_axis_name="s")
    fn = pl.kernel(
        functools.partial(_sc_body, nchunk, ch),
        out_type=jax.ShapeDtypeStruct((n,), jnp.float32),
        mesh=mesh,
        scratch_types=[
            pltpu.VMEM((2, ch), jnp.int32),
            pltpu.VMEM((2, ch), jnp.int32),
            pltpu.VMEM((2, ch), jnp.float32),
            pltpu.VMEM((_LANES,), jnp.float32),
            pltpu.VMEM((_LANES,), jnp.float32),
            pltpu.VMEM((_LANES,), jnp.float32),
            pltpu.SemaphoreType.DMA,
            pltpu.SemaphoreType.DMA,
            pltpu.SemaphoreType.DMA,
            pltpu.SemaphoreType.DMA,
        ],
        compiler_params=pltpu.CompilerParams(needs_layout_passes=False),
    )
    return fn(wpe16, wte16, pos_f, idx_f)


def kernel(pos, idx, wpe, wte):
    b, l = pos.shape
    n = b * l
    per_w = n // _NW
    assert per_w * _NW == n
    ch = per_w
    nchunk = 1
    while 6 * ch * 4 > 400 * 1024:
        for d in (2, 5, 3, 7):
            if ch % d == 0:
                ch //= d
                nchunk *= d
                break
        else:
            raise ValueError("no valid chunking")
    assert ch % _LANES == 0

    pos_f = pos.reshape(-1)
    idx_f = idx.reshape(-1)
    wpe16 = jnp.pad(wpe.reshape(-1), (0, _LANES - wpe.size))
    wte16 = jnp.pad(wte.reshape(-1), (0, _LANES - wte.size))
    out = _sc_lookup(wpe16, wte16, pos_f, idx_f, nchunk, ch)
    return out.reshape(b, l, 1)


# speedup vs baseline: 1.8989x; 1.6877x over previous
import functools

import jax
import jax.numpy as jnp
from jax import lax
from jax.experimental import pallas as pl
from jax.experimental.pallas import tpu as pltpu
from jax.experimental.pallas import tpu_sc as plsc

_NC = 2
_NS = 16
_NW = _NC * _NS
_LANES = 16
_UNROLL = 2
_RCH = 64


def _sc_body(nchunk, l, lpad, wpe_hbm, wte_hbm, pos_hbm, idx_hbm, out_hbm,
             pos_v, idx_v, out_v, wpe_v, wte_v, tab_v,
             is0, is1, os0, os1):
    wid = lax.axis_index("s") * _NC + lax.axis_index("c")
    base_w = wid * nchunk * _RCH
    isems = (is0, is1)
    osems = (os0, os1)
    vec_per_row = l // _LANES

    pltpu.sync_copy(wpe_hbm, wpe_v)
    pltpu.sync_copy(wte_hbm, wte_v)
    lane = lax.iota(jnp.int32, _LANES)
    p_i = lane // 3
    i_i = lane - p_i * 3
    tab_v[...] = (plsc.load_gather(wpe_v, [p_i])
                  + plsc.load_gather(wte_v, [i_i]))

    def in_copies(ci, buf):
        r0 = base_w + ci * _RCH
        return (
            pltpu.make_async_copy(pos_hbm.at[pl.ds(r0, _RCH), :],
                                  pos_v.at[buf], isems[buf]),
            pltpu.make_async_copy(idx_hbm.at[pl.ds(r0, _RCH), :],
                                  idx_v.at[buf], isems[buf]),
        )

    def out_copy(ci, buf):
        r0 = base_w + ci * _RCH
        return pltpu.make_async_copy(out_v.at[buf],
                                     out_hbm.at[pl.ds(r0, _RCH), :],
                                     osems[buf])

    def do_vec(buf, r, c):
        fused = (pos_v[buf, r, pl.ds(c, _LANES)] * 3
                 + idx_v[buf, r, pl.ds(c, _LANES)])
        out_v[buf, r, pl.ds(c, _LANES)] = plsc.load_gather(tab_v, [fused])

    def compute(buf):
        @plsc.parallel_loop(0, _RCH, unroll=_UNROLL)
        def _(r):
            for c in range(0, l - _LANES + 1, _LANES):
                do_vec(buf, r, c)
            if l % _LANES:
                do_vec(buf, r, l - _LANES)

    for c in in_copies(0, 0):
        c.start()
    for ci in range(nchunk):
        buf = ci % 2
        if ci + 1 < nchunk:
            for c in in_copies(ci + 1, 1 - buf):
                c.start()
        for c in in_copies(ci, buf):
            c.wait()
        if ci >= 2:
            out_copy(ci - 2, buf).wait()
        compute(buf)
        out_copy(ci, buf).start()
    for ci in range(max(nchunk - 2, 0), nchunk):
        out_copy(ci, ci % 2).wait()


@functools.partial(jax.jit, static_argnums=(4,))
def _sc_lookup(wpe16, wte16, pos, idx, nchunk):
    b, l = pos.shape
    lpad = (l + _LANES - 1) // _LANES * _LANES
    mesh = plsc.VectorSubcoreMesh(core_axis_name="c", subcore_axis_name="s")
    fn = pl.kernel(
        functools.partial(_sc_body, nchunk, l, lpad),
        out_type=jax.ShapeDtypeStruct((b, l), jnp.float32),
        mesh=mesh,
        scratch_types=[
            pltpu.VMEM((2, _RCH, l), jnp.int32),
            pltpu.VMEM((2, _RCH, l), jnp.int32),
            pltpu.VMEM((2, _RCH, l), jnp.float32),
            pltpu.VMEM((_LANES,), jnp.float32),
            pltpu.VMEM((_LANES,), jnp.float32),
            pltpu.VMEM((_LANES,), jnp.float32),
            pltpu.SemaphoreType.DMA,
            pltpu.SemaphoreType.DMA,
            pltpu.SemaphoreType.DMA,
            pltpu.SemaphoreType.DMA,
        ],
        compiler_params=pltpu.CompilerParams(
            needs_layout_passes=False,
            use_tc_tiling_on_sc=True,
        ),
    )
    return fn(wpe16, wte16, pos, idx)


def kernel(pos, idx, wpe, wte):
    b, l = pos.shape
    rows_w = b // _NW
    assert rows_w * _NW == b and rows_w % _RCH == 0
    nchunk = rows_w // _RCH
    wpe16 = jnp.pad(wpe.reshape(-1), (0, _LANES - wpe.size))
    wte16 = jnp.pad(wte.reshape(-1), (0, _LANES - wte.size))
    out = _sc_lookup(wpe16, wte16, pos, idx, nchunk)
    return out[..., None]
